# baseline (speedup 1.0000x reference)
import functools

import jax
import jax.numpy as jnp
from jax import lax
from jax.experimental import pallas as pl
from jax.experimental.pallas import tpu as pltpu
from jax.experimental.pallas import tpu_sc as plsc

EMBED_D = 64
PAD_D = 128
PAD_H = 56
NBUF = 8
PF = 4


@functools.cache
def _make(batch, hist, nc, ns):
    nw = nc * ns
    b_per_w = batch // nw
    assert b_per_w % NBUF == 0 and b_per_w >= 2 * NBUF
    mesh = plsc.VectorSubcoreMesh(core_axis_name="c", subcore_axis_name="s")

    @functools.partial(
        pl.kernel,
        mesh=mesh,
        out_type=jax.ShapeDtypeStruct((batch, PAD_H, PAD_D), jnp.float32),
        scratch_types=[
            pltpu.VMEM((b_per_w, PAD_D), jnp.int32),
            pltpu.VMEM((NBUF, PAD_H, EMBED_D), jnp.float32),
            [pltpu.SemaphoreType.DMA] * NBUF,
            [pltpu.SemaphoreType.DMA] * NBUF,
        ],
        compiler_params=pltpu.CompilerParams(use_tc_tiling_on_sc=False),
    )
    def k(table_hbm, idx_hbm, out_hbm, idx_v, rows_v, sem_g, sem_w):
        wid = lax.axis_index("s") * nc + lax.axis_index("c")
        base = wid * b_per_w
        pltpu.sync_copy(idx_hbm.at[pl.ds(base, b_per_w)], idx_v)

        def gather(t, b):
            pltpu.make_async_copy(
                table_hbm.at[idx_v.at[t, pl.ds(0, PAD_H)]], rows_v.at[b], sem_g[b]
            ).start()

        def gather_wait(b):
            pltpu.make_async_copy(
                table_hbm.at[idx_v.at[0, pl.ds(0, PAD_H)]], rows_v.at[b], sem_g[b]
            ).wait()

        def writeback(t, b):
            pltpu.make_async_copy(
                rows_v.at[b],
                out_hbm.at[base + t, pl.ds(0, PAD_H), pl.ds(0, EMBED_D)],
                sem_w[b],
            ).start()

        def writeback_wait(b):
            pltpu.make_async_copy(
                rows_v.at[b],
                out_hbm.at[base, pl.ds(0, PAD_H), pl.ds(0, EMBED_D)],
                sem_w[b],
            ).wait()

        for b in range(PF):
            gather(b, b)

        def body(g, carry):
            for b in range(NBUF):
                t = g * NBUF + b
                gather_wait(b)
                writeback(t, b)
                nt = t + PF
                bb = (b + PF) % NBUF

                @pl.when(nt < b_per_w)
                def _():
                    @pl.when(nt >= NBUF)
                    def _():
                        writeback_wait(bb)

                    gather(nt, bb)

            return carry

        lax.fori_loop(0, b_per_w // NBUF, body, 0)

        for b in range(NBUF):
            writeback_wait(b)

    return k


def kernel(move_ids, table):
    batch, hist = move_ids.shape
    info = plsc.get_sparse_core_info()
    nc, ns = info.num_cores, info.num_subcores
    idx_pad = jnp.pad(move_ids.astype(jnp.int32), ((0, 0), (0, PAD_D - hist)))
    out3 = _make(batch, hist, nc, ns)(table, idx_pad)
    return lax.slice(out3, (0, 0, 0), (batch, hist, EMBED_D))


# speedup vs baseline: 4.3646x; 4.3646x over previous
import functools

import jax
import jax.numpy as jnp
from jax import lax
from jax.experimental import pallas as pl
from jax.experimental.pallas import tpu as pltpu
from jax.experimental.pallas import tpu_sc as plsc

EMBED_D = 64
PAD_D = 128
PAD_H = 56
NBUF = 8
PF = 6


@functools.cache
def _make(batch, hist, nc, ns):
    nw = nc * ns
    b_per_w = batch // nw
    assert b_per_w % NBUF == 0 and b_per_w >= 2 * NBUF
    mesh = plsc.VectorSubcoreMesh(core_axis_name="c", subcore_axis_name="s")

    @functools.partial(
        pl.kernel,
        mesh=mesh,
        out_type=jax.ShapeDtypeStruct((batch, PAD_H, PAD_D), jnp.float32),
        scratch_types=[
            pltpu.VMEM((b_per_w, hist), jnp.int32),
            pltpu.VMEM((NBUF, hist, EMBED_D), jnp.float32),
            [pltpu.SemaphoreType.DMA] * NBUF,
            [pltpu.SemaphoreType.DMA] * NBUF,
        ],
        compiler_params=pltpu.CompilerParams(use_tc_tiling_on_sc=False),
    )
    def k(table_hbm, idx_hbm, out_hbm, idx_v, rows_v, sem_g, sem_w):
        wid = lax.axis_index("s") * nc + lax.axis_index("c")
        base = wid * b_per_w
        pltpu.sync_copy(idx_hbm.at[pl.ds(base, b_per_w)], idx_v)

        def gather(t, b):
            pltpu.make_async_copy(
                table_hbm.at[idx_v.at[t]], rows_v.at[b], sem_g[b]
            ).start()

        def gather_wait(b):
            pltpu.make_async_copy(
                table_hbm.at[idx_v.at[0]], rows_v.at[b], sem_g[b]
            ).wait()

        def writeback(t, b):
            pltpu.make_async_copy(
                rows_v.at[b],
                out_hbm.at[base + t, pl.ds(0, hist), pl.ds(0, EMBED_D)],
                sem_w[b],
            ).start()

        def writeback_wait(b):
            pltpu.make_async_copy(
                rows_v.at[b],
                out_hbm.at[base, pl.ds(0, hist), pl.ds(0, EMBED_D)],
                sem_w[b],
            ).wait()

        for b in range(PF):
            gather(b, b)

        def body(g, carry):
            for b in range(NBUF):
                t = g * NBUF + b
                gather_wait(b)
                writeback(t, b)
                nt = t + PF
                bb = (b + PF) % NBUF

                @pl.when(nt < b_per_w)
                def _():
                    @pl.when(nt >= NBUF)
                    def _():
                        writeback_wait(bb)

                    gather(nt, bb)

            return carry

        lax.fori_loop(0, b_per_w // NBUF, body, 0)

        for b in range(NBUF):
            writeback_wait(b)

    return k


def kernel(move_ids, table):
    batch, hist = move_ids.shape
    info = plsc.get_sparse_core_info()
    nc, ns = info.num_cores, info.num_subcores
    out3 = _make(batch, hist, nc, ns)(table, move_ids.astype(jnp.int32))
    return lax.slice(out3, (0, 0, 0), (batch, hist, EMBED_D))


# speedup vs baseline: 4.3752x; 1.0024x over previous
import functools

import jax
import jax.numpy as jnp
from jax import lax
from jax.experimental import pallas as pl
from jax.experimental.pallas import tpu as pltpu
from jax.experimental.pallas import tpu_sc as plsc

EMBED_D = 64
PAD_D = 128
PAD_H = 56
NBUF = 8
PF = 7


@functools.cache
def _make(batch, hist, nc, ns):
    nw = nc * ns
    b_per_w = batch // nw
    assert b_per_w % NBUF == 0 and b_per_w >= 2 * NBUF
    mesh = plsc.VectorSubcoreMesh(core_axis_name="c", subcore_axis_name="s")

    @functools.partial(
        pl.kernel,
        mesh=mesh,
        out_type=jax.ShapeDtypeStruct((batch, PAD_H, PAD_D), jnp.float32),
        scratch_types=[
            pltpu.VMEM((b_per_w, hist), jnp.int32),
            pltpu.VMEM((NBUF, hist, EMBED_D), jnp.float32),
            [pltpu.SemaphoreType.DMA] * NBUF,
            [pltpu.SemaphoreType.DMA] * NBUF,
        ],
        compiler_params=pltpu.CompilerParams(use_tc_tiling_on_sc=False),
    )
    def k(table_hbm, idx_hbm, out_hbm, idx_v, rows_v, sem_g, sem_w):
        wid = lax.axis_index("s") * nc + lax.axis_index("c")
        base = wid * b_per_w
        pltpu.sync_copy(idx_hbm.at[pl.ds(base, b_per_w)], idx_v)

        def gather(t, b):
            pltpu.make_async_copy(
                table_hbm.at[idx_v.at[t]], rows_v.at[b], sem_g[b]
            ).start()

        def gather_wait(b):
            pltpu.make_async_copy(
                table_hbm.at[idx_v.at[0]], rows_v.at[b], sem_g[b]
            ).wait()

        def writeback(t, b):
            pltpu.make_async_copy(
                rows_v.at[b],
                out_hbm.at[base + t, pl.ds(0, hist), pl.ds(0, EMBED_D)],
                sem_w[b],
            ).start()

        def writeback_wait(b):
            pltpu.make_async_copy(
                rows_v.at[b],
                out_hbm.at[base, pl.ds(0, hist), pl.ds(0, EMBED_D)],
                sem_w[b],
            ).wait()

        for b in range(PF):
            gather(b, b)

        def body(g, carry):
            for b in range(NBUF):
                t = g * NBUF + b
                gather_wait(b)
                writeback(t, b)
                nt = t + PF
                bb = (b + PF) % NBUF

                @pl.when(nt < b_per_w)
                def _():
                    @pl.when(nt >= NBUF)
                    def _():
                        writeback_wait(bb)

                    gather(nt, bb)

            return carry

        lax.fori_loop(0, b_per_w // NBUF, body, 0)

        for b in range(NBUF):
            writeback_wait(b)

    return k


def kernel(move_ids, table):
    batch, hist = move_ids.shape
    info = plsc.get_sparse_core_info()
    nc, ns = info.num_cores, info.num_subcores
    out3 = _make(batch, hist, nc, ns)(table, move_ids.astype(jnp.int32))
    return lax.slice(out3, (0, 0, 0), (batch, hist, EMBED_D))
